# baseline (speedup 1.0000x reference)
import functools

import jax
import jax.numpy as jnp
from jax import lax
from jax.experimental import pallas as pl
from jax.experimental.pallas import tpu as pltpu
from jax.experimental.pallas import tpu_sc as plsc

_K = 1


def kernel(x, pos_emb):
    B, T = x.shape
    D = pos_emb.shape[1]
    TD = T * D
    pe = pos_emb[:T].reshape(1, TD)

    info = plsc.get_sparse_core_info()
    nw = info.num_cores * info.num_subcores
    rows_per_w = B // nw
    n_copies = rows_per_w // _K

    mesh = plsc.VectorSubcoreMesh(core_axis_name="c", subcore_axis_name="s")

    @functools.partial(
        pl.kernel,
        mesh=mesh,
        out_type=jax.ShapeDtypeStruct((B, TD), jnp.float32),
        scratch_types=[
            pltpu.VMEM((_K, TD), jnp.float32),
            pltpu.SemaphoreType.DMA,
            pltpu.SemaphoreType.DMA,
        ],
    )
    def sc_broadcast(pe_hbm, out_hbm, buf, lsem, ssem):
        wid = lax.axis_index("s") * info.num_cores + lax.axis_index("c")
        base = wid * rows_per_w
        for i in range(_K):
            pltpu.async_copy(pe_hbm, buf.at[pl.ds(i, 1)], lsem)
        for i in range(_K):
            pltpu.make_async_copy(pe_hbm, buf.at[pl.ds(i, 1)], lsem).wait()
        half = TD // 2
        for j in range(n_copies):
            for h in range(2):
                pltpu.async_copy(
                    buf.at[:, pl.ds(h * half, half)],
                    out_hbm.at[pl.ds(base + j * _K, _K), pl.ds(h * half, half)],
                    ssem,
                )
        for j in range(n_copies):
            for h in range(2):
                pltpu.make_async_copy(
                    buf.at[:, pl.ds(h * half, half)],
                    out_hbm.at[pl.ds(base + j * _K, _K), pl.ds(h * half, half)],
                    ssem,
                ).wait()

    out = sc_broadcast(pe)
    return out.reshape(B, T, D)


# speedup vs baseline: 1.0017x; 1.0017x over previous
import functools

import jax
import jax.numpy as jnp
from jax import lax
from jax.experimental import pallas as pl
from jax.experimental.pallas import tpu as pltpu
from jax.experimental.pallas import tpu_sc as plsc

_K = 1


def kernel(x, pos_emb):
    B, T = x.shape
    D = pos_emb.shape[1]
    TD = T * D
    pe = pos_emb[:T].reshape(1, TD)

    info = plsc.get_sparse_core_info()
    nw = info.num_cores * info.num_subcores
    rows_per_w = B // nw
    n_copies = rows_per_w // _K

    mesh = plsc.VectorSubcoreMesh(core_axis_name="c", subcore_axis_name="s")

    @functools.partial(
        pl.kernel,
        mesh=mesh,
        out_type=jax.ShapeDtypeStruct((B, TD), jnp.float32),
        scratch_types=[
            pltpu.VMEM((_K, TD), jnp.float32),
            pltpu.SemaphoreType.DMA,
            pltpu.SemaphoreType.DMA,
        ],
    )
    def sc_broadcast(pe_hbm, out_hbm, buf, lsem, ssem):
        wid = lax.axis_index("s") * info.num_cores + lax.axis_index("c")
        base = wid * rows_per_w
        for i in range(_K):
            pltpu.async_copy(pe_hbm, buf.at[pl.ds(i, 1)], lsem)
        for i in range(_K):
            pltpu.make_async_copy(pe_hbm, buf.at[pl.ds(i, 1)], lsem).wait()
        for j in range(n_copies):
            pltpu.async_copy(buf, out_hbm.at[pl.ds(base + j * _K, _K)], ssem)
        for j in range(n_copies):
            pltpu.make_async_copy(
                buf, out_hbm.at[pl.ds(base + j * _K, _K)], ssem
            ).wait()

    out = sc_broadcast(pe)
    return out.reshape(B, T, D)


# speedup vs baseline: 1.0039x; 1.0023x over previous
import functools

import jax
import jax.numpy as jnp
from jax import lax
from jax.experimental import pallas as pl
from jax.experimental.pallas import tpu as pltpu
from jax.experimental.pallas import tpu_sc as plsc

_K = 1


def kernel(x, pos_emb):
    B, T = x.shape
    D = pos_emb.shape[1]
    TD = T * D
    pe = pos_emb[:T].reshape(1, TD)

    info = plsc.get_sparse_core_info()
    nw = info.num_cores * info.num_subcores
    rows_per_w = B // nw
    n_copies = rows_per_w // _K

    mesh = plsc.VectorSubcoreMesh(core_axis_name="c", subcore_axis_name="s")

    @functools.partial(
        pl.kernel,
        mesh=mesh,
        out_type=jax.ShapeDtypeStruct((B, TD), jnp.float32),
        scratch_types=[
            pltpu.VMEM((_K, TD), jnp.float32),
            pltpu.SemaphoreType.DMA,
            pltpu.SemaphoreType.DMA((4,)),
        ],
    )
    def sc_broadcast(pe_hbm, out_hbm, buf, lsem, ssem):
        wid = lax.axis_index("s") * info.num_cores + lax.axis_index("c")
        base = wid * rows_per_w
        for i in range(_K):
            pltpu.async_copy(pe_hbm, buf.at[pl.ds(i, 1)], lsem)
        for i in range(_K):
            pltpu.make_async_copy(pe_hbm, buf.at[pl.ds(i, 1)], lsem).wait()
        for j in range(n_copies):
            pltpu.async_copy(
                buf, out_hbm.at[pl.ds(base + j * _K, _K)], ssem.at[j % 4]
            )
        for j in range(n_copies):
            pltpu.make_async_copy(
                buf, out_hbm.at[pl.ds(base + j * _K, _K)], ssem.at[j % 4]
            ).wait()

    out = sc_broadcast(pe)
    return out.reshape(B, T, D)
